# baseline (speedup 1.0000x reference)
import functools

import jax
import jax.numpy as jnp
from jax import lax
from jax.experimental import pallas as pl
from jax.experimental.pallas import tpu as pltpu
from jax.experimental.pallas import tpu_sc as plsc

N = 10000
NNZ = 320000
D = 128

NC = 2
NS = 16
NW = NC * NS
B = 80
NBUF = 4
SLAB = 16
NSEG = 8
CHUNKS = SLAB * NSEG
NNZ_PAD = NW * B * CHUNKS
ROWS_PER_TILE = 640
N_PAD = NS * ROWS_PER_TILE
LANES = 16


def _matmul_body(x_ref, w_ref, b_ref, out_ref):
    out_ref[...] = (
        jnp.dot(x_ref[...], w_ref[...], preferred_element_type=jnp.float32)
        + b_ref[...]
    )


def _combine_body(p_ref, out_ref):
    out_ref[...] = p_ref[0] + p_ref[1]


def _spmm_body(xw_hbm, gi_hbm, vals_hbm, part_hbm,
               rows_v, cols_v, vals_v, g0, g1, g2, g3, acc,
               gsem0, gsem1, gsem2, gsem3, ssem0, ssem1, ssem2, ssem3):
    gbuf = (g0, g1, g2, g3)
    gsem = (gsem0, gsem1, gsem2, gsem3)
    ssem = (ssem0, ssem1, ssem2, ssem3)
    cid = lax.axis_index("c")
    sid = lax.axis_index("s")
    wid = sid * NC + cid
    base = sid * ROWS_PER_TILE

    def _fire_gather(jl, b):
        pltpu.async_copy(xw_hbm.at[cols_v.at[jl]], gbuf[b], gsem[b])

    def _wait_gather(jl, b):
        pltpu.make_async_copy(xw_hbm.at[cols_v.at[jl]], gbuf[b],
                              gsem[b]).wait()

    def _fire_scatter(jl, b):
        pltpu.async_copy(gbuf[b], acc.at[rows_v.at[jl]], ssem[b], add=True)

    def _wait_scatter(jl, b):
        pltpu.make_async_copy(gbuf[b], acc.at[rows_v.at[jl]], ssem[b]).wait()

    def _scale(jl, b):
        @plsc.parallel_loop(0, B // LANES, 1, unroll=2)
        def _group(g):
            vv = vals_v[jl, pl.ds(g * LANES, LANES)]
            for r in range(LANES):
                i = g * LANES + r
                v = vv[r]
                for c in range(D // LANES):
                    sl = (i, pl.ds(c * LANES, LANES))
                    gbuf[b][sl] = gbuf[b][sl] * v

    def _zero_row(r, _):
        for c in range(D // LANES):
            g0[r, pl.ds(c * LANES, LANES)] = jnp.zeros((LANES,), jnp.float32)
        return 0
    lax.fori_loop(0, B, _zero_row, 0)
    for k in range(ROWS_PER_TILE // B):
        pltpu.sync_copy(g0, acc.at[pl.ds(base + k * B, B)])
    if ROWS_PER_TILE % B:
        pltpu.sync_copy(g0.at[pl.ds(0, ROWS_PER_TILE % B)],
                        acc.at[pl.ds(base + (ROWS_PER_TILE // B) * B,
                                     ROWS_PER_TILE % B)])
    plsc.subcore_barrier()

    def _segment(seg, _):
        ra = pltpu.async_copy(gi_hbm.at[0, wid, seg], rows_v, gsem0)
        ca = pltpu.async_copy(gi_hbm.at[1, wid, seg], cols_v, gsem1)
        va = pltpu.async_copy(vals_hbm.at[wid, seg], vals_v, gsem2)
        ra.wait()
        ca.wait()
        va.wait()

        for pb in range(NBUF - 1):
            _fire_gather(pb, pb)

        def _turn(t, _):
            for b in range(NBUF):
                jl = t * NBUF + b
                _wait_gather(jl, b)
                _scale(jl, b)
                _fire_scatter(jl, b)

                @pl.when(jl >= 1)
                def _():
                    _wait_scatter(jl - 1, (b - 1) % NBUF)

                @pl.when(jl + NBUF - 1 <= SLAB - 1)
                def _():
                    _fire_gather(jl + NBUF - 1, (b - 1) % NBUF)
            return 0
        lax.fori_loop(0, SLAB // NBUF, _turn, 0)
        _wait_scatter(SLAB - 1, (SLAB - 1) % NBUF)
        return 0
    lax.fori_loop(0, NSEG, _segment, 0)

    plsc.subcore_barrier()
    pltpu.sync_copy(acc.at[pl.ds(base, ROWS_PER_TILE)],
                    part_hbm.at[cid, pl.ds(base, ROWS_PER_TILE)])


_spmm = functools.partial(
    pl.kernel,
    out_type=jax.ShapeDtypeStruct((NC, N_PAD, D), jnp.float32),
    mesh=plsc.VectorSubcoreMesh(core_axis_name="c", subcore_axis_name="s"),
    scratch_types=[
        pltpu.VMEM((SLAB, B), jnp.int32),
        pltpu.VMEM((SLAB, B), jnp.int32),
        pltpu.VMEM((SLAB, B), jnp.float32),
        pltpu.VMEM((B, D), jnp.float32),
        pltpu.VMEM((B, D), jnp.float32),
        pltpu.VMEM((B, D), jnp.float32),
        pltpu.VMEM((B, D), jnp.float32),
        pltpu.VMEM_SHARED((N_PAD, D), jnp.float32),
        pltpu.SemaphoreType.DMA,
        pltpu.SemaphoreType.DMA,
        pltpu.SemaphoreType.DMA,
        pltpu.SemaphoreType.DMA,
        pltpu.SemaphoreType.DMA,
        pltpu.SemaphoreType.DMA,
        pltpu.SemaphoreType.DMA,
        pltpu.SemaphoreType.DMA,
    ],
)(_spmm_body)


def kernel(x, g_indices, g_values, weight, bias):
    xw = pl.pallas_call(
        _matmul_body,
        out_shape=jax.ShapeDtypeStruct((N, D), jnp.float32),
    )(x, weight, bias.reshape(1, D))

    pad = NNZ_PAD - NNZ
    gi = jnp.pad(g_indices, ((0, 0), (0, pad)),
                 mode="wrap").reshape(2, NW, NSEG, SLAB, B)
    vals = jnp.pad(g_values, (0, pad)).reshape(NW, NSEG, SLAB, B)

    part = _spmm(xw, gi, vals)

    out = pl.pallas_call(
        _combine_body,
        out_shape=jax.ShapeDtypeStruct((N, D), jnp.float32),
        grid=(10,),
        in_specs=[pl.BlockSpec((NC, N // 10, D), lambda i: (0, i, 0))],
        out_specs=pl.BlockSpec((N // 10, D), lambda i: (i, 0)),
    )(part)
    return out


# speedup vs baseline: 1.0853x; 1.0853x over previous
import functools

import jax
import jax.numpy as jnp
from jax import lax
from jax.experimental import pallas as pl
from jax.experimental.pallas import tpu as pltpu
from jax.experimental.pallas import tpu_sc as plsc

N = 10000
NNZ = 320000
D = 128

NC = 2
NS = 16
NW = NC * NS
B = 96
NBUF = 3
SLAB = 21
NSEG = 5
CHUNKS = SLAB * NSEG
NNZ_PAD = NW * B * CHUNKS
ROWS_PER_TILE = 640
N_PAD = NS * ROWS_PER_TILE
LANES = 16


def _matmul_body(x_ref, w_ref, b_ref, out_ref):
    out_ref[...] = (
        jnp.dot(x_ref[...], w_ref[...], preferred_element_type=jnp.float32)
        + b_ref[...]
    )


def _combine_body(p_ref, out_ref):
    out_ref[...] = p_ref[0] + p_ref[1]


def _spmm_body(xw_hbm, gi_hbm, vals_hbm, part_hbm,
               rows_v, cols_v, vals_v, g0, g1, g2, acc,
               gsem0, gsem1, gsem2, ssem0, ssem1, ssem2):
    gbuf = (g0, g1, g2)
    gsem = (gsem0, gsem1, gsem2)
    ssem = (ssem0, ssem1, ssem2)
    cid = lax.axis_index("c")
    sid = lax.axis_index("s")
    wid = sid * NC + cid
    base = sid * ROWS_PER_TILE

    def _fire_gather(jl, b):
        pltpu.async_copy(xw_hbm.at[cols_v.at[jl]], gbuf[b], gsem[b])

    def _wait_gather(jl, b):
        pltpu.make_async_copy(xw_hbm.at[cols_v.at[jl]], gbuf[b],
                              gsem[b]).wait()

    def _fire_scatter(jl, b):
        pltpu.async_copy(gbuf[b], acc.at[rows_v.at[jl]], ssem[b], add=True)

    def _wait_scatter(jl, b):
        pltpu.make_async_copy(gbuf[b], acc.at[rows_v.at[jl]], ssem[b]).wait()

    def _scale(jl, b):
        @plsc.parallel_loop(0, B // LANES, 1, unroll=2)
        def _group(g):
            vv = vals_v[jl, pl.ds(g * LANES, LANES)]
            for r in range(LANES):
                i = g * LANES + r
                v = vv[r]
                for c in range(D // LANES):
                    sl = (i, pl.ds(c * LANES, LANES))
                    gbuf[b][sl] = gbuf[b][sl] * v

    def _zero_row(r, _):
        for c in range(D // LANES):
            g0[r, pl.ds(c * LANES, LANES)] = jnp.zeros((LANES,), jnp.float32)
        return 0
    lax.fori_loop(0, B, _zero_row, 0)
    for k in range(ROWS_PER_TILE // B):
        pltpu.sync_copy(g0, acc.at[pl.ds(base + k * B, B)])
    pltpu.sync_copy(g0.at[pl.ds(0, ROWS_PER_TILE % B)],
                    acc.at[pl.ds(base + (ROWS_PER_TILE // B) * B,
                                 ROWS_PER_TILE % B)])
    plsc.subcore_barrier()

    def _segment(seg, _):
        ra = pltpu.async_copy(gi_hbm.at[0, wid, seg], rows_v, gsem0)
        ca = pltpu.async_copy(gi_hbm.at[1, wid, seg], cols_v, gsem1)
        va = pltpu.async_copy(vals_hbm.at[wid, seg], vals_v, gsem2)
        ra.wait()
        ca.wait()
        va.wait()

        _fire_gather(0, 0)
        _fire_gather(1, 1)

        def _turn(t, _):
            for b in range(NBUF):
                jl = t * NBUF + b
                _wait_gather(jl, b)
                _scale(jl, b)
                _fire_scatter(jl, b)

                @pl.when(jl >= 1)
                def _():
                    _wait_scatter(jl - 1, (b - 1) % NBUF)

                @pl.when(jl + NBUF - 1 <= SLAB - 1)
                def _():
                    _fire_gather(jl + NBUF - 1, (b - 1) % NBUF)
            return 0
        lax.fori_loop(0, SLAB // NBUF, _turn, 0)
        _wait_scatter(SLAB - 1, (SLAB - 1) % NBUF)
        return 0
    lax.fori_loop(0, NSEG, _segment, 0)

    plsc.subcore_barrier()
    pltpu.sync_copy(acc.at[pl.ds(base, ROWS_PER_TILE)],
                    part_hbm.at[cid, pl.ds(base, ROWS_PER_TILE)])


_spmm = functools.partial(
    pl.kernel,
    out_type=jax.ShapeDtypeStruct((NC, N_PAD, D), jnp.float32),
    mesh=plsc.VectorSubcoreMesh(core_axis_name="c", subcore_axis_name="s"),
    scratch_types=[
        pltpu.VMEM((SLAB, B), jnp.int32),
        pltpu.VMEM((SLAB, B), jnp.int32),
        pltpu.VMEM((SLAB, B), jnp.float32),
        pltpu.VMEM((B, D), jnp.float32),
        pltpu.VMEM((B, D), jnp.float32),
        pltpu.VMEM((B, D), jnp.float32),
        pltpu.VMEM_SHARED((N_PAD, D), jnp.float32),
        pltpu.SemaphoreType.DMA,
        pltpu.SemaphoreType.DMA,
        pltpu.SemaphoreType.DMA,
        pltpu.SemaphoreType.DMA,
        pltpu.SemaphoreType.DMA,
        pltpu.SemaphoreType.DMA,
    ],
)(_spmm_body)


def kernel(x, g_indices, g_values, weight, bias):
    xw = pl.pallas_call(
        _matmul_body,
        out_shape=jax.ShapeDtypeStruct((N, D), jnp.float32),
    )(x, weight, bias.reshape(1, D))

    pad = NNZ_PAD - NNZ
    gi = jnp.pad(g_indices, ((0, 0), (0, pad)),
                 mode="wrap").reshape(2, NW, NSEG, SLAB, B)
    vals = jnp.pad(g_values, (0, pad)).reshape(NW, NSEG, SLAB, B)

    part = _spmm(xw, gi, vals)

    out = pl.pallas_call(
        _combine_body,
        out_shape=jax.ShapeDtypeStruct((N, D), jnp.float32),
        grid=(10,),
        in_specs=[pl.BlockSpec((NC, N // 10, D), lambda i: (0, i, 0))],
        out_specs=pl.BlockSpec((N // 10, D), lambda i: (i, 0)),
    )(part)
    return out
